# baseline (speedup 1.0000x reference)
import functools

import jax
import jax.numpy as jnp
from jax import lax
from jax.experimental import pallas as pl
from jax.experimental.pallas import tpu as pltpu
from jax.experimental.pallas import tpu_sc as plsc

_IDX_CH = 128
_LAG = 8


@jax.jit
def kernel(batch, embedding_weight):
    B = batch.shape[0]
    V, D = embedding_weight.shape

    info = plsc.get_sparse_core_info()
    nw = info.num_cores * info.num_subcores
    b_per_w = B // nw
    n_off = b_per_w * D
    half = b_per_w // 2
    h_off = half * D
    h_ch = h_off // _IDX_CH

    idx = batch.astype(jnp.int32)
    table8 = embedding_weight.T.reshape(D * V // 8, 8)
    mesh = plsc.VectorSubcoreMesh(core_axis_name="c", subcore_axis_name="s")

    @functools.partial(
        pl.kernel,
        mesh=mesh,
        out_type=jax.ShapeDtypeStruct((B * D,), jnp.float32),
        compiler_params=pltpu.CompilerParams(
            use_tc_tiling_on_sc=False, needs_layout_passes=False
        ),
        scratch_types=[
            pltpu.VMEM((b_per_w,), jnp.int32),
            pltpu.VMEM((h_off,), jnp.int32),
            pltpu.VMEM((h_off, 8), jnp.float32),
            pltpu.VMEM((n_off,), jnp.float32),
            pltpu.SemaphoreType.DMA,
        ],
    )
    def gather_kernel(tab_hbm, idx_hbm, out_hbm, idx_v, off_v, sl_v, outb_v,
                      sem):
        wid = lax.axis_index("s") * info.num_cores + lax.axis_index("c")
        base = wid * b_per_w
        pltpu.sync_copy(idx_hbm.at[pl.ds(base, b_per_w)], idx_v)

        lanes = lax.iota(jnp.int32, 16)
        jrow0 = lanes * (V // 8)
        jrow1 = (lanes + 16) * (V // 8)
        plane = lanes * D

        for h in range(2):
            def off_step(k16, _):
                ivec = idx_v[pl.ds(h * half + k16 * 16, 16)]
                cvec = lax.shift_right_logical(ivec, 3)
                for u in range(16):
                    kl = k16 * 16 + u
                    off_v[pl.ds(kl * D, 16)] = cvec[u] + jrow0
                    off_v[pl.ds(kl * D + 16, 16)] = cvec[u] + jrow1
                return _

            lax.fori_loop(0, half // 16, off_step, None)

            def gather_step(s, _):
                @pl.when(s < h_ch)
                def _fire():
                    pltpu.async_copy(
                        tab_hbm.at[off_v.at[pl.ds(s * _IDX_CH, _IDX_CH)]],
                        sl_v.at[pl.ds(s * _IDX_CH, _IDX_CH)],
                        sem,
                    )

                @pl.when(s >= _LAG)
                def _drain():
                    pltpu.make_async_copy(
                        tab_hbm.at[off_v.at[pl.ds(0, _IDX_CH)]],
                        sl_v.at[pl.ds(0, _IDX_CH)],
                        sem,
                    ).wait()

                return _

            lax.fori_loop(0, h_ch + _LAG, gather_step, None)

            def ex_step(k16, _):
                ivec = idx_v[pl.ds(h * half + k16 * 16, 16)]
                rvec = lax.bitwise_and(ivec, jnp.full((16,), 7, jnp.int32))
                for u in range(16):
                    kl = k16 * 16 + u
                    rfull = jnp.full((16,), 1, jnp.int32) * rvec[u]
                    v0 = plsc.load_gather(sl_v, [kl * D + lanes, rfull])
                    v1 = plsc.load_gather(sl_v, [kl * D + 16 + lanes, rfull])
                    outb_v[pl.ds((h * half + k16 * 16 + u) * D, 16)] = v0
                    outb_v[pl.ds((h * half + k16 * 16 + u) * D + 16, 16)] = v1
                return _

            lax.fori_loop(0, half // 16, ex_step, None)

        pltpu.sync_copy(outb_v, out_hbm.at[pl.ds(base * D, n_off)])

    return gather_kernel(table8, idx).reshape(B, D)


# speedup vs baseline: 19.2866x; 19.2866x over previous
import functools

import jax
import jax.numpy as jnp
from jax import lax
from jax.experimental import pallas as pl
from jax.experimental.pallas import tpu as pltpu
from jax.experimental.pallas import tpu_sc as plsc

_G = 16


@jax.jit
def kernel(batch, embedding_weight):
    B = batch.shape[0]
    V, D = embedding_weight.shape

    info = plsc.get_sparse_core_info()
    nw = info.num_cores * info.num_subcores
    b_per_w = B // nw
    n_groups = b_per_w // _G

    idx = batch.astype(jnp.int32)
    table_t = embedding_weight.T
    mesh = plsc.VectorSubcoreMesh(core_axis_name="c", subcore_axis_name="s")

    @functools.partial(
        pl.kernel,
        mesh=mesh,
        out_type=jax.ShapeDtypeStruct((D, B), jnp.float32),
        compiler_params=pltpu.CompilerParams(needs_layout_passes=False),
        scratch_types=[
            pltpu.VMEM((b_per_w,), jnp.int32),
            pltpu.VMEM((_G, D, 128), jnp.float32),
            pltpu.VMEM((D, b_per_w), jnp.float32),
            pltpu.SemaphoreType.DMA,
        ],
    )
    def gather_kernel(tab_hbm, idx_hbm, out_hbm, idx_v, win_v, outb_v, sem):
        wid = lax.axis_index("s") * info.num_cores + lax.axis_index("c")
        base = wid * b_per_w
        pltpu.sync_copy(idx_hbm.at[pl.ds(base, b_per_w)], idx_v)

        lanes = lax.iota(jnp.int32, 16)
        mask127 = jnp.full((16,), 127, jnp.int32)

        def group_step(g, _):
            ivec = idx_v[pl.ds(g * _G, _G)]
            avec = lax.shift_left(
                lax.shift_right_logical(ivec, 7), jnp.full((16,), 7, jnp.int32)
            )
            for u in range(_G):
                off = pl.multiple_of(avec[u], 128)
                pltpu.async_copy(
                    tab_hbm.at[:, pl.ds(off, 128)], win_v.at[u], sem
                )
            for _u in range(_G):
                pltpu.make_async_copy(
                    tab_hbm.at[:, pl.ds(0, 128)], win_v.at[0], sem
                ).wait()

            rvec = lax.bitwise_and(ivec, mask127)
            for j in range(D):
                jvec = jnp.full((16,), j, jnp.int32)
                vec = plsc.load_gather(win_v, [lanes, jvec, rvec])
                outb_v[j, pl.ds(g * _G, _G)] = vec
            return _

        lax.fori_loop(0, n_groups, group_step, None)

        pltpu.sync_copy(outb_v, out_hbm.at[:, pl.ds(base, b_per_w)])

    return gather_kernel(table_t, idx).T
